# baseline (speedup 1.0000x reference)
import dataclasses
import functools

import jax
import jax.numpy as jnp
from jax import lax
from jax.experimental import pallas as pl
from jax.experimental.pallas import tpu as pltpu
from jax.experimental.pallas import tpu_sc as plsc

NC = 2
NS = 16
NW = NC * NS
NQ = 4

_HCHUNK = 4000
_ECHUNK = 6400


def _sc_mesh():
    return plsc.VectorSubcoreMesh(core_axis_name="c", subcore_axis_name="s")


def _sc_params():
    cp = pltpu.CompilerParams()
    if "needs_layout_passes" in pltpu.CompilerParams.__dataclass_fields__:
        cp = dataclasses.replace(cp, needs_layout_passes=False)
    return cp


def _deg_hist(dst32, z1):
    np_ = z1.shape[1]
    e = dst32.shape[0]
    epw = e // NW
    nwin = epw // _HCHUNK

    @functools.partial(
        pl.kernel,
        mesh=_sc_mesh(),
        compiler_params=_sc_params(),
        out_type=jax.ShapeDtypeStruct((NW, 1, np_), jnp.float32),
        scratch_types=[
            pltpu.VMEM((1, np_), jnp.float32),
            pltpu.VMEM((_HCHUNK,), jnp.int32),
            pltpu.VMEM((_HCHUNK,), jnp.int32),
            pltpu.SemaphoreType.DMA,
            pltpu.SemaphoreType.DMA,
        ],
    )
    def k(dst_hbm, z1_hbm, deg_out, hist, da, db, sa, sb):
        c = lax.axis_index("c")
        s = lax.axis_index("s")
        wid = s * NC + c

        pltpu.sync_copy(z1_hbm, hist)
        zero16 = jnp.zeros((16,), jnp.int32)
        ones16 = jnp.full((16,), 1.0, jnp.float32)
        base = wid * epw

        def off(w):
            return base + jnp.minimum(w, nwin - 1) * _HCHUNK

        def process(buf):
            @plsc.parallel_loop(0, _HCHUNK, step=16, unroll=10)
            def _(i):
                idx = buf[pl.ds(i, 16)]
                plsc.addupdate_scatter(hist, [zero16, idx], ones16)

        pltpu.async_copy(dst_hbm.at[pl.ds(off(0), _HCHUNK)], da, sa)
        pltpu.async_copy(dst_hbm.at[pl.ds(off(1), _HCHUNK)], db, sb)

        @pl.loop(0, nwin, step=2)
        def _(w):
            pltpu.make_async_copy(dst_hbm.at[pl.ds(0, _HCHUNK)], da, sa).wait()
            process(da)
            pltpu.async_copy(dst_hbm.at[pl.ds(off(w + 2), _HCHUNK)], da, sa)

            @pl.when(w + 1 < nwin)
            def _():
                pltpu.make_async_copy(dst_hbm.at[pl.ds(0, _HCHUNK)], db,
                                      sb).wait()
                process(db)
                pltpu.async_copy(dst_hbm.at[pl.ds(off(w + 3), _HCHUNK)], db,
                                 sb)

        pltpu.make_async_copy(dst_hbm.at[pl.ds(0, _HCHUNK)], da, sa).wait()
        pltpu.make_async_copy(dst_hbm.at[pl.ds(0, _HCHUNK)], db, sb).wait()

        pltpu.sync_copy(hist, deg_out.at[wid])

    return k(dst32, z1)


def _msg_gather(src32, g_t3):
    oc, _, np_ = g_t3.shape
    e = src32.shape[0]
    epq = e // NQ
    nwin = epq // _ECHUNK

    @functools.partial(
        pl.kernel,
        mesh=_sc_mesh(),
        compiler_params=_sc_params(),
        out_type=jax.ShapeDtypeStruct((oc, 1, e), jnp.float32),
        scratch_types=[
            pltpu.VMEM((1, np_), jnp.float32),
            pltpu.VMEM((_ECHUNK,), jnp.int32),
            pltpu.VMEM((_ECHUNK,), jnp.int32),
            pltpu.VMEM((_ECHUNK,), jnp.float32),
            pltpu.VMEM((_ECHUNK,), jnp.float32),
            pltpu.SemaphoreType.DMA,
            pltpu.SemaphoreType.DMA,
            pltpu.SemaphoreType.DMA,
            pltpu.SemaphoreType.DMA,
        ],
    )
    def k(src_hbm, g_hbm, msg_out, g_ch, sa_v, sb_v, ma_v, mb_v,
          isa, isb, osa, osb):
        c = lax.axis_index("c")
        s = lax.axis_index("s")
        wid = s * NC + c
        ch = wid % oc
        q = wid // oc

        pltpu.sync_copy(g_hbm.at[ch], g_ch)
        zero16 = jnp.zeros((16,), jnp.int32)
        base = q * epq

        def off(w):
            return base + jnp.minimum(w, nwin - 1) * _ECHUNK

        def process(src_v, msg_v):
            @plsc.parallel_loop(0, _ECHUNK, step=16, unroll=32)
            def _(i):
                sl = pl.ds(i, 16)
                msg_v[sl] = plsc.load_gather(g_ch, [zero16, src_v[sl]])

        pltpu.async_copy(src_hbm.at[pl.ds(off(0), _ECHUNK)], sa_v, isa)
        pltpu.async_copy(src_hbm.at[pl.ds(off(1), _ECHUNK)], sb_v, isb)

        @pl.loop(0, nwin, step=2)
        def _(w):
            pltpu.make_async_copy(src_hbm.at[pl.ds(0, _ECHUNK)], sa_v,
                                  isa).wait()

            @pl.when(w >= 2)
            def _():
                pltpu.make_async_copy(
                    ma_v, msg_out.at[ch, 0, pl.ds(0, _ECHUNK)], osa).wait()

            process(sa_v, ma_v)
            pltpu.async_copy(ma_v, msg_out.at[ch, 0, pl.ds(off(w), _ECHUNK)],
                             osa)
            pltpu.async_copy(src_hbm.at[pl.ds(off(w + 2), _ECHUNK)], sa_v, isa)

            @pl.when(w + 1 < nwin)
            def _():
                pltpu.make_async_copy(src_hbm.at[pl.ds(0, _ECHUNK)], sb_v,
                                      isb).wait()

                @pl.when(w >= 2)
                def _():
                    pltpu.make_async_copy(
                        mb_v, msg_out.at[ch, 0, pl.ds(0, _ECHUNK)], osb).wait()

                process(sb_v, mb_v)
                pltpu.async_copy(mb_v,
                                 msg_out.at[ch, 0, pl.ds(off(w + 1), _ECHUNK)],
                                 osb)
                pltpu.async_copy(src_hbm.at[pl.ds(off(w + 3), _ECHUNK)], sb_v,
                                 isb)

        pltpu.make_async_copy(src_hbm.at[pl.ds(0, _ECHUNK)], sa_v, isa).wait()
        pltpu.make_async_copy(src_hbm.at[pl.ds(0, _ECHUNK)], sb_v, isb).wait()
        pltpu.make_async_copy(ma_v, msg_out.at[ch, 0, pl.ds(0, _ECHUNK)],
                              osa).wait()
        pltpu.make_async_copy(mb_v, msg_out.at[ch, 0, pl.ds(0, _ECHUNK)],
                              osb).wait()

    return k(src32, g_t3)


def _msg_scatter(dst32, msg3, z1):
    np_ = z1.shape[1]
    oc, _, e = msg3.shape
    epq = e // NQ
    nwin = epq // _ECHUNK

    @functools.partial(
        pl.kernel,
        mesh=_sc_mesh(),
        compiler_params=_sc_params(),
        out_type=jax.ShapeDtypeStruct((NQ, oc, 1, np_), jnp.float32),
        scratch_types=[
            pltpu.VMEM((1, np_), jnp.float32),
            pltpu.VMEM((_ECHUNK,), jnp.int32),
            pltpu.VMEM((_ECHUNK,), jnp.int32),
            pltpu.VMEM((_ECHUNK,), jnp.float32),
            pltpu.VMEM((_ECHUNK,), jnp.float32),
            pltpu.SemaphoreType.DMA,
            pltpu.SemaphoreType.DMA,
            pltpu.SemaphoreType.DMA,
            pltpu.SemaphoreType.DMA,
        ],
    )
    def k(dst_hbm, msg_hbm, z1_hbm, acc_out, acc, da_v, db_v, ma_v, mb_v,
          ida, idb, ima, imb):
        c = lax.axis_index("c")
        s = lax.axis_index("s")
        wid = s * NC + c
        ch = wid % oc
        q = wid // oc

        pltpu.sync_copy(z1_hbm, acc)
        zero16 = jnp.zeros((16,), jnp.int32)
        base = q * epq

        def off(w):
            return base + jnp.minimum(w, nwin - 1) * _ECHUNK

        def process(dst_v, msg_v):
            @plsc.parallel_loop(0, _ECHUNK, step=16, unroll=32)
            def _(i):
                sl = pl.ds(i, 16)
                plsc.addupdate_scatter(acc, [zero16, dst_v[sl]], msg_v[sl])

        pltpu.async_copy(dst_hbm.at[pl.ds(off(0), _ECHUNK)], da_v, ida)
        pltpu.async_copy(msg_hbm.at[ch, 0, pl.ds(off(0), _ECHUNK)], ma_v, ima)
        pltpu.async_copy(dst_hbm.at[pl.ds(off(1), _ECHUNK)], db_v, idb)
        pltpu.async_copy(msg_hbm.at[ch, 0, pl.ds(off(1), _ECHUNK)], mb_v, imb)

        @pl.loop(0, nwin, step=2)
        def _(w):
            pltpu.make_async_copy(dst_hbm.at[pl.ds(0, _ECHUNK)], da_v,
                                  ida).wait()
            pltpu.make_async_copy(msg_hbm.at[ch, 0, pl.ds(0, _ECHUNK)], ma_v,
                                  ima).wait()
            process(da_v, ma_v)
            pltpu.async_copy(dst_hbm.at[pl.ds(off(w + 2), _ECHUNK)], da_v, ida)
            pltpu.async_copy(msg_hbm.at[ch, 0, pl.ds(off(w + 2), _ECHUNK)],
                             ma_v, ima)

            @pl.when(w + 1 < nwin)
            def _():
                pltpu.make_async_copy(dst_hbm.at[pl.ds(0, _ECHUNK)], db_v,
                                      idb).wait()
                pltpu.make_async_copy(msg_hbm.at[ch, 0, pl.ds(0, _ECHUNK)],
                                      mb_v, imb).wait()
                process(db_v, mb_v)
                pltpu.async_copy(dst_hbm.at[pl.ds(off(w + 3), _ECHUNK)], db_v,
                                 idb)
                pltpu.async_copy(msg_hbm.at[ch, 0, pl.ds(off(w + 3), _ECHUNK)],
                                 mb_v, imb)

        pltpu.make_async_copy(dst_hbm.at[pl.ds(0, _ECHUNK)], da_v, ida).wait()
        pltpu.make_async_copy(dst_hbm.at[pl.ds(0, _ECHUNK)], db_v, idb).wait()
        pltpu.make_async_copy(msg_hbm.at[ch, 0, pl.ds(0, _ECHUNK)], ma_v,
                              ima).wait()
        pltpu.make_async_copy(msg_hbm.at[ch, 0, pl.ds(0, _ECHUNK)], mb_v,
                              imb).wait()

        pltpu.sync_copy(acc, acc_out.at[q, ch])

    return k(dst32, msg3, z1)


def _tc_g(x, w_t, deg_parts, blk):
    n, ic = x.shape
    oc = w_t.shape[0]
    np_ = deg_parts.shape[2]

    def body(deg_ref, x_ref, w_ref, g_ref):
        deg = jnp.sum(deg_ref[...], axis=(0, 1)) + 1.0
        dinv = lax.rsqrt(deg)
        h = lax.dot_general(w_ref[...], x_ref[...],
                            (((1,), (1,)), ((), ())),
                            preferred_element_type=jnp.float32)
        g_ref[...] = (h * dinv[None, :]).reshape(oc, 1, blk)

    return pl.pallas_call(
        body,
        grid=(np_ // blk,),
        in_specs=[
            pl.BlockSpec((NW, 1, blk), lambda i: (0, 0, i)),
            pl.BlockSpec((blk, ic), lambda i: (i, 0)),
            pl.BlockSpec((oc, ic), lambda i: (0, 0)),
        ],
        out_specs=pl.BlockSpec((oc, 1, blk), lambda i: (0, 0, i)),
        out_shape=jax.ShapeDtypeStruct((oc, 1, np_), jnp.float32),
    )(deg_parts, x, w_t)


def _tc_out(acc_parts, g_t3, deg_parts, b_c, n, blk):
    oc = g_t3.shape[0]
    np_ = g_t3.shape[2]

    def body(a_ref, g_ref, deg_ref, b_ref, o_ref):
        deg = jnp.sum(deg_ref[...], axis=(0, 1)) + 1.0
        dinv = lax.rsqrt(deg)
        ssum = jnp.sum(a_ref[...], axis=(0, 2)) + g_ref[:, 0, :]
        o_ref[...] = ssum * dinv[None, :] + b_ref[...]

    return pl.pallas_call(
        body,
        grid=((n + blk - 1) // blk,),
        in_specs=[
            pl.BlockSpec((NQ, oc, 1, blk), lambda i: (0, 0, 0, i)),
            pl.BlockSpec((oc, 1, blk), lambda i: (0, 0, i)),
            pl.BlockSpec((NW, 1, blk), lambda i: (0, 0, i)),
            pl.BlockSpec((oc, 1), lambda i: (0, 0)),
        ],
        out_specs=pl.BlockSpec((oc, blk), lambda i: (0, i)),
        out_shape=jax.ShapeDtypeStruct((oc, n), jnp.float32),
    )(acc_parts, g_t3, deg_parts, b_c)


def kernel(x, edge_index, W, b):
    n = x.shape[0]
    oc = W.shape[1]
    np_ = 128 * ((n + 127) // 128)
    blk = 4352

    src32 = edge_index[0].astype(jnp.int32)
    dst32 = edge_index[1].astype(jnp.int32)
    w_t = W.T
    b_c = b.reshape(oc, 1)
    z1 = jnp.zeros((1, np_), jnp.float32)

    deg_parts = _deg_hist(dst32, z1)
    g_t3 = _tc_g(x, w_t, deg_parts, blk)
    msg3 = _msg_gather(src32, g_t3)
    acc4 = _msg_scatter(dst32, msg3, z1)
    out_t = _tc_out(acc4, g_t3, deg_parts, b_c, n, blk)
    return out_t.T


# speedup vs baseline: 1.0312x; 1.0312x over previous
import dataclasses
import functools

import jax
import jax.numpy as jnp
from jax import lax
from jax.experimental import pallas as pl
from jax.experimental.pallas import tpu as pltpu
from jax.experimental.pallas import tpu_sc as plsc

NC = 2
NS = 16
NW = NC * NS
NQ = 4

_HCHUNK = 4000
_ECHUNK = 6400


def _sc_mesh():
    return plsc.VectorSubcoreMesh(core_axis_name="c", subcore_axis_name="s")


def _sc_params():
    cp = pltpu.CompilerParams()
    if "needs_layout_passes" in pltpu.CompilerParams.__dataclass_fields__:
        cp = dataclasses.replace(cp, needs_layout_passes=False)
    return cp


def _deg_hist(dst32, z1):
    np_ = z1.shape[1]
    e = dst32.shape[0]
    epw = e // NW
    nwin = epw // _HCHUNK

    @functools.partial(
        pl.kernel,
        mesh=_sc_mesh(),
        compiler_params=_sc_params(),
        out_type=jax.ShapeDtypeStruct((NW, 1, np_), jnp.float32),
        scratch_types=[
            pltpu.VMEM((1, np_), jnp.float32),
            pltpu.VMEM((_HCHUNK,), jnp.int32),
            pltpu.VMEM((_HCHUNK,), jnp.int32),
            pltpu.SemaphoreType.DMA,
            pltpu.SemaphoreType.DMA,
        ],
    )
    def k(dst_hbm, z1_hbm, deg_out, hist, da, db, sa, sb):
        c = lax.axis_index("c")
        s = lax.axis_index("s")
        wid = s * NC + c

        pltpu.sync_copy(z1_hbm, hist)
        zero16 = jnp.zeros((16,), jnp.int32)
        ones16 = jnp.full((16,), 1.0, jnp.float32)
        base = wid * epw

        def off(w):
            return base + jnp.minimum(w, nwin - 1) * _HCHUNK

        def process(buf):
            @plsc.parallel_loop(0, _HCHUNK, step=16, unroll=10)
            def _(i):
                idx = buf[pl.ds(i, 16)]
                plsc.addupdate_scatter(hist, [zero16, idx], ones16)

        pltpu.async_copy(dst_hbm.at[pl.ds(off(0), _HCHUNK)], da, sa)
        pltpu.async_copy(dst_hbm.at[pl.ds(off(1), _HCHUNK)], db, sb)

        @pl.loop(0, nwin, step=2)
        def _(w):
            pltpu.make_async_copy(dst_hbm.at[pl.ds(0, _HCHUNK)], da, sa).wait()
            process(da)
            pltpu.async_copy(dst_hbm.at[pl.ds(off(w + 2), _HCHUNK)], da, sa)

            @pl.when(w + 1 < nwin)
            def _():
                pltpu.make_async_copy(dst_hbm.at[pl.ds(0, _HCHUNK)], db,
                                      sb).wait()
                process(db)
                pltpu.async_copy(dst_hbm.at[pl.ds(off(w + 3), _HCHUNK)], db,
                                 sb)

        pltpu.make_async_copy(dst_hbm.at[pl.ds(0, _HCHUNK)], da, sa).wait()
        pltpu.make_async_copy(dst_hbm.at[pl.ds(0, _HCHUNK)], db, sb).wait()

        pltpu.sync_copy(hist, deg_out.at[wid])

    return k(dst32, z1)


def _msg_gather(src32, g_t3):
    oc, _, np_ = g_t3.shape
    e = src32.shape[0]
    epq = e // NQ
    nwin = epq // _ECHUNK

    @functools.partial(
        pl.kernel,
        mesh=_sc_mesh(),
        compiler_params=_sc_params(),
        out_type=jax.ShapeDtypeStruct((oc, 1, e), jnp.float32),
        scratch_types=[
            pltpu.VMEM((1, np_), jnp.float32),
            pltpu.VMEM((_ECHUNK,), jnp.int32),
            pltpu.VMEM((_ECHUNK,), jnp.int32),
            pltpu.VMEM((_ECHUNK,), jnp.float32),
            pltpu.VMEM((_ECHUNK,), jnp.float32),
            pltpu.SemaphoreType.DMA,
            pltpu.SemaphoreType.DMA,
            pltpu.SemaphoreType.DMA,
            pltpu.SemaphoreType.DMA,
        ],
    )
    def k(src_hbm, g_hbm, msg_out, g_ch, sa_v, sb_v, ma_v, mb_v,
          isa, isb, osa, osb):
        c = lax.axis_index("c")
        s = lax.axis_index("s")
        wid = s * NC + c
        ch = wid % oc
        q = wid // oc

        pltpu.sync_copy(g_hbm.at[ch], g_ch)
        zero16 = jnp.zeros((16,), jnp.int32)
        base = q * epq

        def off(w):
            return base + jnp.minimum(w, nwin - 1) * _ECHUNK

        def process(src_v, msg_v):
            @plsc.parallel_loop(0, _ECHUNK, step=16, unroll=16)
            def _(i):
                sl = pl.ds(i, 16)
                msg_v[sl] = plsc.load_gather(g_ch, [zero16, src_v[sl]])

        pltpu.async_copy(src_hbm.at[pl.ds(off(0), _ECHUNK)], sa_v, isa)
        pltpu.async_copy(src_hbm.at[pl.ds(off(1), _ECHUNK)], sb_v, isb)

        @pl.loop(0, nwin, step=2)
        def _(w):
            pltpu.make_async_copy(src_hbm.at[pl.ds(0, _ECHUNK)], sa_v,
                                  isa).wait()

            @pl.when(w >= 2)
            def _():
                pltpu.make_async_copy(
                    ma_v, msg_out.at[ch, 0, pl.ds(0, _ECHUNK)], osa).wait()

            process(sa_v, ma_v)
            pltpu.async_copy(ma_v, msg_out.at[ch, 0, pl.ds(off(w), _ECHUNK)],
                             osa)
            pltpu.async_copy(src_hbm.at[pl.ds(off(w + 2), _ECHUNK)], sa_v, isa)

            @pl.when(w + 1 < nwin)
            def _():
                pltpu.make_async_copy(src_hbm.at[pl.ds(0, _ECHUNK)], sb_v,
                                      isb).wait()

                @pl.when(w >= 2)
                def _():
                    pltpu.make_async_copy(
                        mb_v, msg_out.at[ch, 0, pl.ds(0, _ECHUNK)], osb).wait()

                process(sb_v, mb_v)
                pltpu.async_copy(mb_v,
                                 msg_out.at[ch, 0, pl.ds(off(w + 1), _ECHUNK)],
                                 osb)
                pltpu.async_copy(src_hbm.at[pl.ds(off(w + 3), _ECHUNK)], sb_v,
                                 isb)

        pltpu.make_async_copy(src_hbm.at[pl.ds(0, _ECHUNK)], sa_v, isa).wait()
        pltpu.make_async_copy(src_hbm.at[pl.ds(0, _ECHUNK)], sb_v, isb).wait()
        pltpu.make_async_copy(ma_v, msg_out.at[ch, 0, pl.ds(0, _ECHUNK)],
                              osa).wait()
        pltpu.make_async_copy(mb_v, msg_out.at[ch, 0, pl.ds(0, _ECHUNK)],
                              osb).wait()

    return k(src32, g_t3)


def _msg_scatter(dst32, msg3, z1):
    np_ = z1.shape[1]
    oc, _, e = msg3.shape
    epq = e // NQ
    nwin = epq // _ECHUNK

    @functools.partial(
        pl.kernel,
        mesh=_sc_mesh(),
        compiler_params=_sc_params(),
        out_type=jax.ShapeDtypeStruct((NQ, oc, 1, np_), jnp.float32),
        scratch_types=[
            pltpu.VMEM((1, np_), jnp.float32),
            pltpu.VMEM((_ECHUNK,), jnp.int32),
            pltpu.VMEM((_ECHUNK,), jnp.int32),
            pltpu.VMEM((_ECHUNK,), jnp.float32),
            pltpu.VMEM((_ECHUNK,), jnp.float32),
            pltpu.SemaphoreType.DMA,
            pltpu.SemaphoreType.DMA,
            pltpu.SemaphoreType.DMA,
            pltpu.SemaphoreType.DMA,
        ],
    )
    def k(dst_hbm, msg_hbm, z1_hbm, acc_out, acc, da_v, db_v, ma_v, mb_v,
          ida, idb, ima, imb):
        c = lax.axis_index("c")
        s = lax.axis_index("s")
        wid = s * NC + c
        ch = wid % oc
        q = wid // oc

        pltpu.sync_copy(z1_hbm, acc)
        zero16 = jnp.zeros((16,), jnp.int32)
        base = q * epq

        def off(w):
            return base + jnp.minimum(w, nwin - 1) * _ECHUNK

        def process(dst_v, msg_v):
            @plsc.parallel_loop(0, _ECHUNK, step=16, unroll=16)
            def _(i):
                sl = pl.ds(i, 16)
                plsc.addupdate_scatter(acc, [zero16, dst_v[sl]], msg_v[sl])

        pltpu.async_copy(dst_hbm.at[pl.ds(off(0), _ECHUNK)], da_v, ida)
        pltpu.async_copy(msg_hbm.at[ch, 0, pl.ds(off(0), _ECHUNK)], ma_v, ima)
        pltpu.async_copy(dst_hbm.at[pl.ds(off(1), _ECHUNK)], db_v, idb)
        pltpu.async_copy(msg_hbm.at[ch, 0, pl.ds(off(1), _ECHUNK)], mb_v, imb)

        @pl.loop(0, nwin, step=2)
        def _(w):
            pltpu.make_async_copy(dst_hbm.at[pl.ds(0, _ECHUNK)], da_v,
                                  ida).wait()
            pltpu.make_async_copy(msg_hbm.at[ch, 0, pl.ds(0, _ECHUNK)], ma_v,
                                  ima).wait()
            process(da_v, ma_v)
            pltpu.async_copy(dst_hbm.at[pl.ds(off(w + 2), _ECHUNK)], da_v, ida)
            pltpu.async_copy(msg_hbm.at[ch, 0, pl.ds(off(w + 2), _ECHUNK)],
                             ma_v, ima)

            @pl.when(w + 1 < nwin)
            def _():
                pltpu.make_async_copy(dst_hbm.at[pl.ds(0, _ECHUNK)], db_v,
                                      idb).wait()
                pltpu.make_async_copy(msg_hbm.at[ch, 0, pl.ds(0, _ECHUNK)],
                                      mb_v, imb).wait()
                process(db_v, mb_v)
                pltpu.async_copy(dst_hbm.at[pl.ds(off(w + 3), _ECHUNK)], db_v,
                                 idb)
                pltpu.async_copy(msg_hbm.at[ch, 0, pl.ds(off(w + 3), _ECHUNK)],
                                 mb_v, imb)

        pltpu.make_async_copy(dst_hbm.at[pl.ds(0, _ECHUNK)], da_v, ida).wait()
        pltpu.make_async_copy(dst_hbm.at[pl.ds(0, _ECHUNK)], db_v, idb).wait()
        pltpu.make_async_copy(msg_hbm.at[ch, 0, pl.ds(0, _ECHUNK)], ma_v,
                              ima).wait()
        pltpu.make_async_copy(msg_hbm.at[ch, 0, pl.ds(0, _ECHUNK)], mb_v,
                              imb).wait()

        pltpu.sync_copy(acc, acc_out.at[q, ch])

    return k(dst32, msg3, z1)


def _tc_g(x, w_t, deg_parts, blk):
    n, ic = x.shape
    oc = w_t.shape[0]
    np_ = deg_parts.shape[2]

    def body(deg_ref, x_ref, w_ref, g_ref):
        deg = jnp.sum(deg_ref[...], axis=(0, 1)) + 1.0
        dinv = lax.rsqrt(deg)
        h = lax.dot_general(w_ref[...], x_ref[...],
                            (((1,), (1,)), ((), ())),
                            preferred_element_type=jnp.float32)
        g_ref[...] = (h * dinv[None, :]).reshape(oc, 1, blk)

    return pl.pallas_call(
        body,
        grid=(np_ // blk,),
        in_specs=[
            pl.BlockSpec((NW, 1, blk), lambda i: (0, 0, i)),
            pl.BlockSpec((blk, ic), lambda i: (i, 0)),
            pl.BlockSpec((oc, ic), lambda i: (0, 0)),
        ],
        out_specs=pl.BlockSpec((oc, 1, blk), lambda i: (0, 0, i)),
        out_shape=jax.ShapeDtypeStruct((oc, 1, np_), jnp.float32),
    )(deg_parts, x, w_t)


def _tc_out(acc_parts, g_t3, deg_parts, b_c, n, blk):
    oc = g_t3.shape[0]
    np_ = g_t3.shape[2]

    def body(a_ref, g_ref, deg_ref, b_ref, o_ref):
        deg = jnp.sum(deg_ref[...], axis=(0, 1)) + 1.0
        dinv = lax.rsqrt(deg)
        ssum = jnp.sum(a_ref[...], axis=(0, 2)) + g_ref[:, 0, :]
        o_ref[...] = ssum * dinv[None, :] + b_ref[...]

    return pl.pallas_call(
        body,
        grid=((n + blk - 1) // blk,),
        in_specs=[
            pl.BlockSpec((NQ, oc, 1, blk), lambda i: (0, 0, 0, i)),
            pl.BlockSpec((oc, 1, blk), lambda i: (0, 0, i)),
            pl.BlockSpec((NW, 1, blk), lambda i: (0, 0, i)),
            pl.BlockSpec((oc, 1), lambda i: (0, 0)),
        ],
        out_specs=pl.BlockSpec((oc, blk), lambda i: (0, i)),
        out_shape=jax.ShapeDtypeStruct((oc, n), jnp.float32),
    )(acc_parts, g_t3, deg_parts, b_c)


def kernel(x, edge_index, W, b):
    n = x.shape[0]
    oc = W.shape[1]
    np_ = 128 * ((n + 127) // 128)
    blk = 4352

    src32 = edge_index[0].astype(jnp.int32)
    dst32 = edge_index[1].astype(jnp.int32)
    w_t = W.T
    b_c = b.reshape(oc, 1)
    z1 = jnp.zeros((1, np_), jnp.float32)

    deg_parts = _deg_hist(dst32, z1)
    g_t3 = _tc_g(x, w_t, deg_parts, blk)
    msg3 = _msg_gather(src32, g_t3)
    acc4 = _msg_scatter(dst32, msg3, z1)
    out_t = _tc_out(acc4, g_t3, deg_parts, b_c, n, blk)
    return out_t.T


# speedup vs baseline: 1.0836x; 1.0508x over previous
import dataclasses
import functools

import jax
import jax.numpy as jnp
from jax import lax
from jax.experimental import pallas as pl
from jax.experimental.pallas import tpu as pltpu
from jax.experimental.pallas import tpu_sc as plsc

NC = 2
NS = 16
NW = NC * NS
NQ = 4

_HCHUNK = 4000
_ECHUNK = 6400


def _sc_mesh():
    return plsc.VectorSubcoreMesh(core_axis_name="c", subcore_axis_name="s")


def _sc_params():
    cp = pltpu.CompilerParams()
    if "needs_layout_passes" in pltpu.CompilerParams.__dataclass_fields__:
        cp = dataclasses.replace(cp, needs_layout_passes=False)
    return cp


def _deg_hist(dst32, z1):
    np_ = z1.shape[1]
    e = dst32.shape[0]
    epw = e // NW
    nwin = epw // _HCHUNK

    @functools.partial(
        pl.kernel,
        mesh=_sc_mesh(),
        compiler_params=_sc_params(),
        out_type=jax.ShapeDtypeStruct((NW, 1, np_), jnp.float32),
        scratch_types=[
            pltpu.VMEM((1, np_), jnp.float32),
            pltpu.VMEM((_HCHUNK,), jnp.int32),
            pltpu.VMEM((_HCHUNK,), jnp.int32),
            pltpu.SemaphoreType.DMA,
            pltpu.SemaphoreType.DMA,
        ],
    )
    def k(dst_hbm, z1_hbm, deg_out, hist, da, db, sa, sb):
        c = lax.axis_index("c")
        s = lax.axis_index("s")
        wid = s * NC + c

        pltpu.sync_copy(z1_hbm, hist)
        zero16 = jnp.zeros((16,), jnp.int32)
        ones16 = jnp.full((16,), 1.0, jnp.float32)
        base = wid * epw

        def off(w):
            return base + jnp.minimum(w, nwin - 1) * _HCHUNK

        def process(buf):
            @plsc.parallel_loop(0, _HCHUNK, step=16, unroll=10)
            def _(i):
                idx = buf[pl.ds(i, 16)]
                plsc.addupdate_scatter(hist, [zero16, idx], ones16)

        pltpu.async_copy(dst_hbm.at[pl.ds(off(0), _HCHUNK)], da, sa)
        pltpu.async_copy(dst_hbm.at[pl.ds(off(1), _HCHUNK)], db, sb)

        @pl.loop(0, nwin, step=2)
        def _(w):
            pltpu.make_async_copy(dst_hbm.at[pl.ds(0, _HCHUNK)], da, sa).wait()
            process(da)
            pltpu.async_copy(dst_hbm.at[pl.ds(off(w + 2), _HCHUNK)], da, sa)

            @pl.when(w + 1 < nwin)
            def _():
                pltpu.make_async_copy(dst_hbm.at[pl.ds(0, _HCHUNK)], db,
                                      sb).wait()
                process(db)
                pltpu.async_copy(dst_hbm.at[pl.ds(off(w + 3), _HCHUNK)], db,
                                 sb)

        pltpu.make_async_copy(dst_hbm.at[pl.ds(0, _HCHUNK)], da, sa).wait()
        pltpu.make_async_copy(dst_hbm.at[pl.ds(0, _HCHUNK)], db, sb).wait()

        pltpu.sync_copy(hist, deg_out.at[wid])

    return k(dst32, z1)


def _msg_gather(src32, g_t3):
    oc, _, np_ = g_t3.shape
    e = src32.shape[0]
    epq = e // NQ
    nwin = epq // _ECHUNK

    @functools.partial(
        pl.kernel,
        mesh=_sc_mesh(),
        compiler_params=_sc_params(),
        out_type=jax.ShapeDtypeStruct((oc, 1, e), jnp.float32),
        scratch_types=[
            pltpu.VMEM((1, np_), jnp.float32),
            pltpu.VMEM((_ECHUNK,), jnp.int32),
            pltpu.VMEM((_ECHUNK,), jnp.int32),
            pltpu.VMEM((_ECHUNK,), jnp.float32),
            pltpu.VMEM((_ECHUNK,), jnp.float32),
            pltpu.SemaphoreType.DMA,
            pltpu.SemaphoreType.DMA,
            pltpu.SemaphoreType.DMA,
            pltpu.SemaphoreType.DMA,
        ],
    )
    def k(src_hbm, g_hbm, msg_out, g_ch, sa_v, sb_v, ma_v, mb_v,
          isa, isb, osa, osb):
        c = lax.axis_index("c")
        s = lax.axis_index("s")
        wid = s * NC + c
        ch = wid % oc
        q = wid // oc

        pltpu.sync_copy(g_hbm.at[ch], g_ch)
        zero16 = jnp.zeros((16,), jnp.int32)
        base = q * epq

        def off(w):
            return base + jnp.minimum(w, nwin - 1) * _ECHUNK

        def process(src_v, msg_v):
            @plsc.parallel_loop(0, _ECHUNK, step=16, unroll=16)
            def _(i):
                sl = pl.ds(i, 16)
                msg_v[sl] = plsc.load_gather(g_ch, [zero16, src_v[sl]])

        pltpu.async_copy(src_hbm.at[pl.ds(off(0), _ECHUNK)], sa_v, isa)
        pltpu.async_copy(src_hbm.at[pl.ds(off(1), _ECHUNK)], sb_v, isb)

        @pl.loop(0, nwin, step=2)
        def _(w):
            pltpu.make_async_copy(src_hbm.at[pl.ds(0, _ECHUNK)], sa_v,
                                  isa).wait()

            @pl.when(w >= 2)
            def _():
                pltpu.make_async_copy(
                    ma_v, msg_out.at[ch, 0, pl.ds(0, _ECHUNK)], osa).wait()

            process(sa_v, ma_v)
            pltpu.async_copy(ma_v, msg_out.at[ch, 0, pl.ds(off(w), _ECHUNK)],
                             osa)
            pltpu.async_copy(src_hbm.at[pl.ds(off(w + 2), _ECHUNK)], sa_v, isa)

            @pl.when(w + 1 < nwin)
            def _():
                pltpu.make_async_copy(src_hbm.at[pl.ds(0, _ECHUNK)], sb_v,
                                      isb).wait()

                @pl.when(w >= 2)
                def _():
                    pltpu.make_async_copy(
                        mb_v, msg_out.at[ch, 0, pl.ds(0, _ECHUNK)], osb).wait()

                process(sb_v, mb_v)
                pltpu.async_copy(mb_v,
                                 msg_out.at[ch, 0, pl.ds(off(w + 1), _ECHUNK)],
                                 osb)
                pltpu.async_copy(src_hbm.at[pl.ds(off(w + 3), _ECHUNK)], sb_v,
                                 isb)

        pltpu.make_async_copy(src_hbm.at[pl.ds(0, _ECHUNK)], sa_v, isa).wait()
        pltpu.make_async_copy(src_hbm.at[pl.ds(0, _ECHUNK)], sb_v, isb).wait()
        pltpu.make_async_copy(ma_v, msg_out.at[ch, 0, pl.ds(0, _ECHUNK)],
                              osa).wait()
        pltpu.make_async_copy(mb_v, msg_out.at[ch, 0, pl.ds(0, _ECHUNK)],
                              osb).wait()

    return k(src32, g_t3)


def _msg_scatter(dst32, msg3, z1):
    np_ = z1.shape[1]
    oc, _, e = msg3.shape
    epq = e // NQ
    nwin = epq // _ECHUNK

    @functools.partial(
        pl.kernel,
        mesh=_sc_mesh(),
        compiler_params=_sc_params(),
        out_type=jax.ShapeDtypeStruct((NQ, oc, 1, np_), jnp.float32),
        scratch_types=[
            pltpu.VMEM((1, np_), jnp.float32),
            pltpu.VMEM((_ECHUNK,), jnp.int32),
            pltpu.VMEM((_ECHUNK,), jnp.int32),
            pltpu.VMEM((_ECHUNK,), jnp.float32),
            pltpu.VMEM((_ECHUNK,), jnp.float32),
            pltpu.SemaphoreType.DMA,
            pltpu.SemaphoreType.DMA,
            pltpu.SemaphoreType.DMA,
            pltpu.SemaphoreType.DMA,
        ],
    )
    def k(dst_hbm, msg_hbm, z1_hbm, acc_out, acc, da_v, db_v, ma_v, mb_v,
          ida, idb, ima, imb):
        c = lax.axis_index("c")
        s = lax.axis_index("s")
        wid = s * NC + c
        ch = wid % oc
        q = wid // oc

        pltpu.sync_copy(z1_hbm, acc)
        zero16 = jnp.zeros((16,), jnp.int32)
        base = q * epq

        def off(w):
            return base + jnp.minimum(w, nwin - 1) * _ECHUNK

        def process(dst_v, msg_v):
            @plsc.parallel_loop(0, _ECHUNK, step=16, unroll=16)
            def _(i):
                sl = pl.ds(i, 16)
                plsc.addupdate_scatter(acc, [zero16, dst_v[sl]], msg_v[sl])

        pltpu.async_copy(dst_hbm.at[pl.ds(off(0), _ECHUNK)], da_v, ida)
        pltpu.async_copy(msg_hbm.at[ch, 0, pl.ds(off(0), _ECHUNK)], ma_v, ima)
        pltpu.async_copy(dst_hbm.at[pl.ds(off(1), _ECHUNK)], db_v, idb)
        pltpu.async_copy(msg_hbm.at[ch, 0, pl.ds(off(1), _ECHUNK)], mb_v, imb)

        @pl.loop(0, nwin, step=2)
        def _(w):
            pltpu.make_async_copy(dst_hbm.at[pl.ds(0, _ECHUNK)], da_v,
                                  ida).wait()
            pltpu.make_async_copy(msg_hbm.at[ch, 0, pl.ds(0, _ECHUNK)], ma_v,
                                  ima).wait()
            process(da_v, ma_v)
            pltpu.async_copy(dst_hbm.at[pl.ds(off(w + 2), _ECHUNK)], da_v, ida)
            pltpu.async_copy(msg_hbm.at[ch, 0, pl.ds(off(w + 2), _ECHUNK)],
                             ma_v, ima)

            @pl.when(w + 1 < nwin)
            def _():
                pltpu.make_async_copy(dst_hbm.at[pl.ds(0, _ECHUNK)], db_v,
                                      idb).wait()
                pltpu.make_async_copy(msg_hbm.at[ch, 0, pl.ds(0, _ECHUNK)],
                                      mb_v, imb).wait()
                process(db_v, mb_v)
                pltpu.async_copy(dst_hbm.at[pl.ds(off(w + 3), _ECHUNK)], db_v,
                                 idb)
                pltpu.async_copy(msg_hbm.at[ch, 0, pl.ds(off(w + 3), _ECHUNK)],
                                 mb_v, imb)

        pltpu.make_async_copy(dst_hbm.at[pl.ds(0, _ECHUNK)], da_v, ida).wait()
        pltpu.make_async_copy(dst_hbm.at[pl.ds(0, _ECHUNK)], db_v, idb).wait()
        pltpu.make_async_copy(msg_hbm.at[ch, 0, pl.ds(0, _ECHUNK)], ma_v,
                              ima).wait()
        pltpu.make_async_copy(msg_hbm.at[ch, 0, pl.ds(0, _ECHUNK)], mb_v,
                              imb).wait()

        pltpu.sync_copy(acc, acc_out.at[q, ch])

    return k(dst32, msg3, z1)


def _tc_g(x_t, w_t, deg_parts, blk):
    ic, n = x_t.shape
    oc = w_t.shape[0]
    np_ = deg_parts.shape[2]

    def body(deg_ref, x_ref, w_ref, g_ref):
        deg = jnp.sum(deg_ref[...], axis=(0, 1)) + 1.0
        dinv = lax.rsqrt(deg)
        h = jnp.dot(w_ref[...], x_ref[...],
                    preferred_element_type=jnp.float32)
        g_ref[...] = (h * dinv[None, :]).reshape(oc, 1, blk)

    return pl.pallas_call(
        body,
        grid=(np_ // blk,),
        in_specs=[
            pl.BlockSpec((NW, 1, blk), lambda i: (0, 0, i)),
            pl.BlockSpec((ic, blk), lambda i: (0, i)),
            pl.BlockSpec((oc, ic), lambda i: (0, 0)),
        ],
        out_specs=pl.BlockSpec((oc, 1, blk), lambda i: (0, 0, i)),
        out_shape=jax.ShapeDtypeStruct((oc, 1, np_), jnp.float32),
    )(deg_parts, x_t, w_t)


def _tc_out(acc_parts, g_t3, deg_parts, b_c, n, blk):
    oc = g_t3.shape[0]
    np_ = g_t3.shape[2]

    def body(a_ref, g_ref, deg_ref, b_ref, o_ref):
        deg = jnp.sum(deg_ref[...], axis=(0, 1)) + 1.0
        dinv = lax.rsqrt(deg)
        ssum = jnp.sum(a_ref[...], axis=(0, 2)) + g_ref[:, 0, :]
        o_ref[...] = ssum * dinv[None, :] + b_ref[...]

    return pl.pallas_call(
        body,
        grid=((n + blk - 1) // blk,),
        in_specs=[
            pl.BlockSpec((NQ, oc, 1, blk), lambda i: (0, 0, 0, i)),
            pl.BlockSpec((oc, 1, blk), lambda i: (0, 0, i)),
            pl.BlockSpec((NW, 1, blk), lambda i: (0, 0, i)),
            pl.BlockSpec((oc, 1), lambda i: (0, 0)),
        ],
        out_specs=pl.BlockSpec((oc, blk), lambda i: (0, i)),
        out_shape=jax.ShapeDtypeStruct((oc, n), jnp.float32),
    )(acc_parts, g_t3, deg_parts, b_c)


def kernel(x, edge_index, W, b):
    n = x.shape[0]
    oc = W.shape[1]
    np_ = 128 * ((n + 127) // 128)
    blk = 4352

    src32 = edge_index[0].astype(jnp.int32)
    dst32 = edge_index[1].astype(jnp.int32)
    x_t = x.T
    w_t = W.T
    b_c = b.reshape(oc, 1)
    z1 = jnp.zeros((1, np_), jnp.float32)

    deg_parts = _deg_hist(dst32, z1)
    g_t3 = _tc_g(x_t, w_t, deg_parts, blk)
    msg3 = _msg_gather(src32, g_t3)
    acc4 = _msg_scatter(dst32, msg3, z1)
    out_t = _tc_out(acc4, g_t3, deg_parts, b_c, n, blk)
    return out_t.T


# speedup vs baseline: 1.1011x; 1.0162x over previous
import dataclasses
import functools

import jax
import jax.numpy as jnp
from jax import lax
from jax.experimental import pallas as pl
from jax.experimental.pallas import tpu as pltpu
from jax.experimental.pallas import tpu_sc as plsc

NC = 2
NS = 16
NW = NC * NS
NQ = 4

_HCHUNK = 4000
_ECHUNK = 6400


def _sc_mesh():
    return plsc.VectorSubcoreMesh(core_axis_name="c", subcore_axis_name="s")


def _sc_params():
    cp = pltpu.CompilerParams()
    if "needs_layout_passes" in pltpu.CompilerParams.__dataclass_fields__:
        cp = dataclasses.replace(cp, needs_layout_passes=False)
    return cp


def _deg_hist(dst32, z1):
    np_ = z1.shape[1]
    e = dst32.shape[0]
    epw = e // NW
    nwin = epw // _HCHUNK

    @functools.partial(
        pl.kernel,
        mesh=_sc_mesh(),
        compiler_params=_sc_params(),
        out_type=jax.ShapeDtypeStruct((NW, 1, np_), jnp.float32),
        scratch_types=[
            pltpu.VMEM((1, np_), jnp.float32),
            pltpu.VMEM((_HCHUNK,), jnp.int32),
            pltpu.VMEM((_HCHUNK,), jnp.int32),
            pltpu.SemaphoreType.DMA,
            pltpu.SemaphoreType.DMA,
        ],
    )
    def k(dst_hbm, z1_hbm, deg_out, hist, da, db, sa, sb):
        c = lax.axis_index("c")
        s = lax.axis_index("s")
        wid = s * NC + c

        pltpu.sync_copy(z1_hbm, hist)
        zero16 = jnp.zeros((16,), jnp.int32)
        ones16 = jnp.full((16,), 1.0, jnp.float32)
        base = wid * epw

        def off(w):
            return base + jnp.minimum(w, nwin - 1) * _HCHUNK

        def process(buf):
            @plsc.parallel_loop(0, _HCHUNK, step=16, unroll=10)
            def _(i):
                idx = buf[pl.ds(i, 16)]
                plsc.addupdate_scatter(hist, [zero16, idx], ones16)

        pltpu.async_copy(dst_hbm.at[pl.ds(off(0), _HCHUNK)], da, sa)
        pltpu.async_copy(dst_hbm.at[pl.ds(off(1), _HCHUNK)], db, sb)

        @pl.loop(0, nwin, step=2)
        def _(w):
            pltpu.make_async_copy(dst_hbm.at[pl.ds(0, _HCHUNK)], da, sa).wait()
            process(da)
            pltpu.async_copy(dst_hbm.at[pl.ds(off(w + 2), _HCHUNK)], da, sa)

            @pl.when(w + 1 < nwin)
            def _():
                pltpu.make_async_copy(dst_hbm.at[pl.ds(0, _HCHUNK)], db,
                                      sb).wait()
                process(db)
                pltpu.async_copy(dst_hbm.at[pl.ds(off(w + 3), _HCHUNK)], db,
                                 sb)

        pltpu.make_async_copy(dst_hbm.at[pl.ds(0, _HCHUNK)], da, sa).wait()
        pltpu.make_async_copy(dst_hbm.at[pl.ds(0, _HCHUNK)], db, sb).wait()

        pltpu.sync_copy(hist, deg_out.at[wid])

    return k(dst32, z1)


def _edge_pass(src32, dst32, g_t3, z1):
    oc, _, np_ = g_t3.shape
    e = src32.shape[0]
    epq = e // NQ
    nwin = epq // _ECHUNK

    @functools.partial(
        pl.kernel,
        mesh=_sc_mesh(),
        compiler_params=_sc_params(),
        out_type=[
            jax.ShapeDtypeStruct((NQ, oc, 1, np_), jnp.float32),
            jax.ShapeDtypeStruct((oc, 1, e), jnp.float32),
        ],
        scratch_types=[
            pltpu.VMEM((1, np_), jnp.float32),
            pltpu.VMEM((_ECHUNK,), jnp.int32),
            pltpu.VMEM((_ECHUNK,), jnp.int32),
            pltpu.VMEM((_ECHUNK,), jnp.float32),
            pltpu.VMEM((_ECHUNK,), jnp.float32),
            pltpu.SemaphoreType.DMA,
            pltpu.SemaphoreType.DMA,
            pltpu.SemaphoreType.DMA,
            pltpu.SemaphoreType.DMA,
        ],
    )
    def k(src_hbm, dst_hbm, g_hbm, z1_hbm, acc_out, msg_hbm,
          buf, sa_v, sb_v, ma_v, mb_v, isa, isb, osa, osb):
        c = lax.axis_index("c")
        s = lax.axis_index("s")
        wid = s * NC + c
        ch = wid % oc
        q = wid // oc

        pltpu.sync_copy(g_hbm.at[ch], buf)
        zero16 = jnp.zeros((16,), jnp.int32)
        base = q * epq

        def off(w):
            return base + jnp.minimum(w, nwin - 1) * _ECHUNK

        def gath(src_v, msg_v):
            @plsc.parallel_loop(0, _ECHUNK, step=16, unroll=16)
            def _(i):
                sl = pl.ds(i, 16)
                msg_v[sl] = plsc.load_gather(buf, [zero16, src_v[sl]])

        pltpu.async_copy(src_hbm.at[pl.ds(off(0), _ECHUNK)], sa_v, isa)
        pltpu.async_copy(src_hbm.at[pl.ds(off(1), _ECHUNK)], sb_v, isb)

        @pl.loop(0, nwin, step=2)
        def _(w):
            pltpu.make_async_copy(src_hbm.at[pl.ds(0, _ECHUNK)], sa_v,
                                  isa).wait()

            @pl.when(w >= 2)
            def _():
                pltpu.make_async_copy(
                    ma_v, msg_hbm.at[ch, 0, pl.ds(0, _ECHUNK)], osa).wait()

            gath(sa_v, ma_v)
            pltpu.async_copy(ma_v, msg_hbm.at[ch, 0, pl.ds(off(w), _ECHUNK)],
                             osa)
            pltpu.async_copy(src_hbm.at[pl.ds(off(w + 2), _ECHUNK)], sa_v, isa)

            @pl.when(w + 1 < nwin)
            def _():
                pltpu.make_async_copy(src_hbm.at[pl.ds(0, _ECHUNK)], sb_v,
                                      isb).wait()

                @pl.when(w >= 2)
                def _():
                    pltpu.make_async_copy(
                        mb_v, msg_hbm.at[ch, 0, pl.ds(0, _ECHUNK)], osb).wait()

                gath(sb_v, mb_v)
                pltpu.async_copy(mb_v,
                                 msg_hbm.at[ch, 0, pl.ds(off(w + 1), _ECHUNK)],
                                 osb)
                pltpu.async_copy(src_hbm.at[pl.ds(off(w + 3), _ECHUNK)], sb_v,
                                 isb)

        pltpu.make_async_copy(src_hbm.at[pl.ds(0, _ECHUNK)], sa_v, isa).wait()
        pltpu.make_async_copy(src_hbm.at[pl.ds(0, _ECHUNK)], sb_v, isb).wait()
        pltpu.make_async_copy(ma_v, msg_hbm.at[ch, 0, pl.ds(0, _ECHUNK)],
                              osa).wait()
        pltpu.make_async_copy(mb_v, msg_hbm.at[ch, 0, pl.ds(0, _ECHUNK)],
                              osb).wait()

        pltpu.sync_copy(z1_hbm, buf)

        def scat(dst_v, msg_v):
            @plsc.parallel_loop(0, _ECHUNK, step=16, unroll=16)
            def _(i):
                sl = pl.ds(i, 16)
                plsc.addupdate_scatter(buf, [zero16, dst_v[sl]], msg_v[sl])

        pltpu.async_copy(dst_hbm.at[pl.ds(off(0), _ECHUNK)], sa_v, isa)
        pltpu.async_copy(msg_hbm.at[ch, 0, pl.ds(off(0), _ECHUNK)], ma_v, osa)
        pltpu.async_copy(dst_hbm.at[pl.ds(off(1), _ECHUNK)], sb_v, isb)
        pltpu.async_copy(msg_hbm.at[ch, 0, pl.ds(off(1), _ECHUNK)], mb_v, osb)

        @pl.loop(0, nwin, step=2)
        def _(w):
            pltpu.make_async_copy(dst_hbm.at[pl.ds(0, _ECHUNK)], sa_v,
                                  isa).wait()
            pltpu.make_async_copy(msg_hbm.at[ch, 0, pl.ds(0, _ECHUNK)], ma_v,
                                  osa).wait()
            scat(sa_v, ma_v)
            pltpu.async_copy(dst_hbm.at[pl.ds(off(w + 2), _ECHUNK)], sa_v, isa)
            pltpu.async_copy(msg_hbm.at[ch, 0, pl.ds(off(w + 2), _ECHUNK)],
                             ma_v, osa)

            @pl.when(w + 1 < nwin)
            def _():
                pltpu.make_async_copy(dst_hbm.at[pl.ds(0, _ECHUNK)], sb_v,
                                      isb).wait()
                pltpu.make_async_copy(msg_hbm.at[ch, 0, pl.ds(0, _ECHUNK)],
                                      mb_v, osb).wait()
                scat(sb_v, mb_v)
                pltpu.async_copy(dst_hbm.at[pl.ds(off(w + 3), _ECHUNK)], sb_v,
                                 isb)
                pltpu.async_copy(msg_hbm.at[ch, 0, pl.ds(off(w + 3), _ECHUNK)],
                                 mb_v, osb)

        pltpu.make_async_copy(dst_hbm.at[pl.ds(0, _ECHUNK)], sa_v, isa).wait()
        pltpu.make_async_copy(dst_hbm.at[pl.ds(0, _ECHUNK)], sb_v, isb).wait()
        pltpu.make_async_copy(msg_hbm.at[ch, 0, pl.ds(0, _ECHUNK)], ma_v,
                              osa).wait()
        pltpu.make_async_copy(msg_hbm.at[ch, 0, pl.ds(0, _ECHUNK)], mb_v,
                              osb).wait()

        pltpu.sync_copy(buf, acc_out.at[q, ch])

    return k(src32, dst32, g_t3, z1)[0]


def _tc_g(x_t, w_t, deg_parts, blk):
    ic, n = x_t.shape
    oc = w_t.shape[0]
    np_ = deg_parts.shape[2]

    def body(deg_ref, x_ref, w_ref, g_ref):
        deg = jnp.sum(deg_ref[...], axis=(0, 1)) + 1.0
        dinv = lax.rsqrt(deg)
        h = jnp.dot(w_ref[...], x_ref[...],
                    preferred_element_type=jnp.float32)
        g_ref[...] = (h * dinv[None, :]).reshape(oc, 1, blk)

    return pl.pallas_call(
        body,
        grid=(np_ // blk,),
        in_specs=[
            pl.BlockSpec((NW, 1, blk), lambda i: (0, 0, i)),
            pl.BlockSpec((ic, blk), lambda i: (0, i)),
            pl.BlockSpec((oc, ic), lambda i: (0, 0)),
        ],
        out_specs=pl.BlockSpec((oc, 1, blk), lambda i: (0, 0, i)),
        out_shape=jax.ShapeDtypeStruct((oc, 1, np_), jnp.float32),
    )(deg_parts, x_t, w_t)


def _tc_out(acc_parts, g_t3, deg_parts, b_c, n, blk):
    oc = g_t3.shape[0]
    np_ = g_t3.shape[2]

    def body(a_ref, g_ref, deg_ref, b_ref, o_ref):
        deg = jnp.sum(deg_ref[...], axis=(0, 1)) + 1.0
        dinv = lax.rsqrt(deg)
        ssum = jnp.sum(a_ref[...], axis=(0, 2)) + g_ref[:, 0, :]
        o_ref[...] = ssum * dinv[None, :] + b_ref[...]

    return pl.pallas_call(
        body,
        grid=((n + blk - 1) // blk,),
        in_specs=[
            pl.BlockSpec((NQ, oc, 1, blk), lambda i: (0, 0, 0, i)),
            pl.BlockSpec((oc, 1, blk), lambda i: (0, 0, i)),
            pl.BlockSpec((NW, 1, blk), lambda i: (0, 0, i)),
            pl.BlockSpec((oc, 1), lambda i: (0, 0)),
        ],
        out_specs=pl.BlockSpec((oc, blk), lambda i: (0, i)),
        out_shape=jax.ShapeDtypeStruct((oc, n), jnp.float32),
    )(acc_parts, g_t3, deg_parts, b_c)


def kernel(x, edge_index, W, b):
    n = x.shape[0]
    oc = W.shape[1]
    np_ = 128 * ((n + 127) // 128)
    blk = 4352

    src32 = edge_index[0].astype(jnp.int32)
    dst32 = edge_index[1].astype(jnp.int32)
    x_t = x.T
    w_t = W.T
    b_c = b.reshape(oc, 1)
    z1 = jnp.zeros((1, np_), jnp.float32)

    deg_parts = _deg_hist(dst32, z1)
    g_t3 = _tc_g(x_t, w_t, deg_parts, blk)
    acc4 = _edge_pass(src32, dst32, g_t3, z1)
    out_t = _tc_out(acc4, g_t3, deg_parts, b_c, n, blk)
    return out_t.T
